# Initial kernel scaffold; baseline (speedup 1.0000x reference)
import jax
import jax.numpy as jnp
from jax.experimental import pallas as pl


def kernel(input_ids, word_embeddings, position_embeddings, token_type_embeddings, gamma, beta):
    raise NotImplementedError("write your pallas kernel here")


# speedup vs baseline: 1.5669x; 1.5669x over previous
import functools

import jax
import jax.numpy as jnp
from jax import lax
from jax.experimental import pallas as pl
from jax.experimental.pallas import tpu as pltpu
from jax.experimental.pallas import tpu_sc as plsc

EMB = 128
LANES = 16
NVEC = EMB // LANES
NC, NS = 2, 16
NW = NC * NS
EPS = 1e-12
MAGIC = 0x5F3759DF


def _sc_body(rows_w, seq_w, ids_hbm, word_hbm, pos_hbm, tt_hbm, gamma_hbm,
             beta_hbm, out_hbm, idx_v, rows_v, pos_v, tt_v, g_v, b_v, gsem):
    wid = lax.axis_index("s") * NC + lax.axis_index("c")
    idx_chunks = rows_w // 128

    pltpu.sync_copy(ids_hbm.at[pl.ds(wid * idx_chunks, idx_chunks)], idx_v)

    cps = [
        pltpu.async_copy(word_hbm.at[idx_v.at[t]],
                         rows_v.at[pl.ds(t * 128, 128)], gsem)
        for t in range(idx_chunks)
    ]
    pos_base = (wid % (seq_w // rows_w)) * rows_w if seq_w > rows_w else 0
    pltpu.sync_copy(pos_hbm.at[pl.ds(pos_base, rows_w)], pos_v)
    pltpu.sync_copy(tt_hbm.at[0], tt_v)
    pltpu.sync_copy(gamma_hbm, g_v)
    pltpu.sync_copy(beta_hbm, b_v)
    for c in cps:
        c.wait()

    tt = [tt_v[pl.ds(j * LANES, LANES)] for j in range(NVEC)]
    gm = [g_v[pl.ds(j * LANES, LANES)] for j in range(NVEC)]
    bt = [b_v[pl.ds(j * LANES, LANES)] for j in range(NVEC)]

    lane = lax.iota(jnp.int32, LANES)
    bfly = [lane ^ k for k in (8, 4, 2, 1)]

    def _allsum(x):
        for idx in bfly:
            x = x + x.at[idx].get(mode="promise_in_bounds")
        return x

    def row(i, carry):
        xs = []
        s1 = jnp.zeros((LANES,), jnp.float32)
        s2 = jnp.zeros((LANES,), jnp.float32)
        for j in range(NVEC):
            w = rows_v[i, pl.ds(j * LANES, LANES)]
            p = pos_v[i, pl.ds(j * LANES, LANES)]
            x = (w + p) + tt[j]
            xs.append(x)
            s1 = s1 + x
            s2 = s2 + x * x
        mu = _allsum(s1) * (1.0 / EMB)
        var = _allsum(s2) * (1.0 / EMB) - mu * mu + EPS
        bits = lax.bitcast_convert_type(var, jnp.int32)
        r = lax.bitcast_convert_type(MAGIC - (bits >> 1), jnp.float32)
        for _ in range(3):
            r = r * (1.5 - 0.5 * var * r * r)
        for j in range(NVEC):
            rows_v[i, pl.ds(j * LANES, LANES)] = (xs[j] - mu) * r * gm[j] + bt[j]
        return carry

    lax.fori_loop(0, rows_w, row, None)
    pltpu.sync_copy(rows_v, out_hbm.at[pl.ds(wid * rows_w, rows_w)])


def kernel(input_ids, word_embeddings, position_embeddings,
           token_type_embeddings, gamma, beta):
    batch, seq = input_ids.shape
    total = batch * seq
    rows_w = total // NW
    idx_chunks = rows_w // 128
    ids2d = input_ids.reshape(total // 128, 128).astype(jnp.int32)

    mesh = plsc.VectorSubcoreMesh(core_axis_name="c", subcore_axis_name="s")
    call = pl.kernel(
        functools.partial(_sc_body, rows_w, seq),
        out_type=jax.ShapeDtypeStruct((total, EMB), jnp.float32),
        mesh=mesh,
        scratch_types=[
            pltpu.VMEM((idx_chunks, 128), jnp.int32),
            pltpu.VMEM((rows_w, EMB), jnp.float32),
            pltpu.VMEM((rows_w, EMB), jnp.float32),
            pltpu.VMEM((EMB,), jnp.float32),
            pltpu.VMEM((EMB,), jnp.float32),
            pltpu.VMEM((EMB,), jnp.float32),
            pltpu.SemaphoreType.DMA,
        ],
    )
    out = call(ids2d, word_embeddings, position_embeddings,
               token_type_embeddings, gamma, beta)
    return out.reshape(batch, seq, EMB)
